# baseline (speedup 1.0000x reference)
import functools

import jax
import jax.numpy as jnp
from jax import lax
from jax.experimental import pallas as pl
from jax.experimental.pallas import tpu as pltpu
from jax.experimental.pallas import tpu_sc as plsc

NC = 2
NS = 16
NW = NC * NS
CH = 128
FIRE = 8


def _swish(v):
  return v * jax.nn.sigmoid(v)


def _pick_tile(total, target):
  for t in range(min(target, total), 0, -8):
    if total % t == 0:
      return t
  return total


def _mesh():
  return plsc.VectorSubcoreMesh(
      core_axis_name="c", subcore_axis_name="s",
      num_cores=NC, num_subcores=NS)


_SC_PARAMS = pltpu.CompilerParams(use_tc_tiling_on_sc=False,
                                 needs_layout_passes=False)


def _make_gather_pair(n, e, d, interpret=False):
  fire = 4
  ew = e // NW
  j = ew // CH
  no = j // fire

  @functools.partial(
      pl.kernel,
      out_type=jax.ShapeDtypeStruct((e, 2 * d), jnp.float32),
      mesh=_mesh(),
      scratch_types=[
          pltpu.VMEM((j, CH), jnp.int32),
          pltpu.VMEM((j, CH), jnp.int32),
          pltpu.VMEM((fire * CH, d), jnp.float32),
          pltpu.VMEM((fire * CH, d), jnp.float32),
          pltpu.SemaphoreType.DMA,
      ],
      compiler_params=_SC_PARAMS,
      interpret=interpret,
  )
  def gk(ta, ia, tb, ib, out, iva, ivb, ba, bb, sem):
    wid = lax.axis_index("s") * NC + lax.axis_index("c")
    base = wid * ew
    pltpu.sync_copy(ia.at[wid], iva)
    pltpu.sync_copy(ib.at[wid], ivb)

    def body(o, carry):
      descs = []
      for b in range(fire):
        descs.append(pltpu.async_copy(
            ta.at[iva.at[o * fire + b]], ba.at[pl.ds(b * CH, CH), :], sem))
        descs.append(pltpu.async_copy(
            tb.at[ivb.at[o * fire + b]], bb.at[pl.ds(b * CH, CH), :], sem))
      for dsc in descs:
        dsc.wait()
      r = base + o * (fire * CH)
      pltpu.sync_copy(ba, out.at[pl.ds(r, fire * CH), pl.ds(0, d)])
      pltpu.sync_copy(bb, out.at[pl.ds(r, fire * CH), pl.ds(d, d)])
      return carry

    lax.fori_loop(0, no, body, 0)

  return gk


def _make_gather_planes(n, e, interpret=False):
  fire = 8
  ew = e // NW
  j = ew // CH
  no = j // fire

  @functools.partial(
      pl.kernel,
      out_type=jax.ShapeDtypeStruct((8, e), jnp.float32),
      mesh=_mesh(),
      scratch_types=[
          pltpu.VMEM((j, CH), jnp.int32),
          pltpu.VMEM((fire * CH, 4), jnp.float32),
          pltpu.VMEM((fire * CH, 4), jnp.float32),
          pltpu.VMEM((8, fire * CH), jnp.float32),
          pltpu.SemaphoreType.DMA,
      ],
      compiler_params=_SC_PARAMS,
      interpret=interpret,
  )
  def gk(t0, t1, idx, out, iv, b0, b1, pb, sem):
    wid = lax.axis_index("s") * NC + lax.axis_index("c")
    base = wid * ew
    pltpu.sync_copy(idx.at[wid], iv)
    iota = lax.iota(jnp.int32, 16)

    def body(o, carry):
      descs = []
      for b in range(fire):
        descs.append(pltpu.async_copy(
            t0.at[iv.at[o * fire + b]], b0.at[pl.ds(b * CH, CH), :], sem))
        descs.append(pltpu.async_copy(
            t1.at[iv.at[o * fire + b]], b1.at[pl.ds(b * CH, CH), :], sem))
      for dsc in descs:
        dsc.wait()

      def rep(g, carry2):
        r = g * 16 + iota
        for p in range(4):
          cols = jnp.full((16,), p, jnp.int32)
          pb[p, pl.ds(g * 16, 16)] = plsc.load_gather(b0, [r, cols])
          pb[p + 4, pl.ds(g * 16, 16)] = plsc.load_gather(b1, [r, cols])
        return carry2

      lax.fori_loop(0, fire * CH // 16, rep, 0)
      for p in range(8):
        pltpu.sync_copy(pb.at[p],
                        out.at[p, pl.ds(base + o * fire * CH, fire * CH)])
      return carry

    lax.fori_loop(0, no, body, 0)

  return gk


def _zero_spmem(vb, sp, s, nrows, chunk, d2):
  def zb(i, carry):
    for q in range(d2 // 16):
      vb[i, pl.ds(q * 16, 16)] = jnp.zeros((16,), jnp.float32)
    return carry

  lax.fori_loop(0, chunk, zb, 0)
  full = nrows // chunk
  for kk in range(full):
    pltpu.sync_copy(vb, sp.at[pl.ds(s * nrows + kk * chunk, chunk), :])
  rem = nrows - full * chunk
  if rem:
    pltpu.sync_copy(vb.at[pl.ds(0, rem), :],
                    sp.at[pl.ds(s * nrows + full * chunk, rem), :])


def _make_scatter_ev(n, e, interpret=False):
  fire = 4
  et = e // NW
  j = et // CH
  no = j // fire
  nrows = n // NS
  grp = fire * CH // 16

  @functools.partial(
      pl.kernel,
      out_type=jax.ShapeDtypeStruct((NC, n, 4), jnp.float32),
      mesh=_mesh(),
      scratch_types=[
          pltpu.VMEM((fire, CH), jnp.int32),
          pltpu.VMEM((fire * CH // 128, 128), jnp.float32),
          pltpu.VMEM((fire * CH // 128, 128), jnp.float32),
          pltpu.VMEM((fire * CH // 128, 128), jnp.float32),
          pltpu.VMEM((fire * CH, 4), jnp.float32),
          pltpu.VMEM_SHARED((n, 4), jnp.float32),
          pltpu.SemaphoreType.DMA,
          pltpu.SemaphoreType.DMA,
      ],
      compiler_params=_SC_PARAMS,
      interpret=interpret,
  )
  def sk(evx, evy, evz, idx, out, iv, px, py, pz, vb, sp, sem, sem2):
    c = lax.axis_index("c")
    s = lax.axis_index("s")
    iota = lax.iota(jnp.int32, 16)
    ones16 = jnp.ones((16,), jnp.float32)
    zeros16 = jnp.zeros((16,), jnp.float32)

    def zb(g, carry):
      r = g * 16 + iota
      for p in range(4):
        plsc.store_scatter(vb, [r, jnp.full((16,), p, jnp.int32)], zeros16)
      return carry

    lax.fori_loop(0, fire * CH // 16, zb, 0)
    full = nrows // (fire * CH)
    for kk in range(full):
      pltpu.sync_copy(vb, sp.at[pl.ds(s * nrows + kk * fire * CH, fire * CH), :])
    rem = nrows - full * fire * CH
    if rem:
      pltpu.sync_copy(vb.at[pl.ds(0, rem), :],
                      sp.at[pl.ds(s * nrows + full * fire * CH, rem), :])
    plsc.subcore_barrier()
    base = (c * NS + s) * et

    def body(o, carry):
      q0 = (base + o * fire * CH) // 128
      qr = fire * CH // 128
      d1 = pltpu.async_copy(idx.at[c, s, pl.ds(o * fire, fire), :], iv, sem2)
      dx = pltpu.async_copy(evx.at[pl.ds(q0, qr), :], px, sem)
      dy = pltpu.async_copy(evy.at[pl.ds(q0, qr), :], py, sem)
      dz = pltpu.async_copy(evz.at[pl.ds(q0, qr), :], pz, sem)
      d1.wait()
      dx.wait()
      dy.wait()
      dz.wait()

      def rep(g, carry2):
        r = g * 16 + iota
        gr = g // 8
        gl = (g % 8) * 16
        plsc.store_scatter(vb, [r, jnp.full((16,), 0, jnp.int32)],
                           px[gr, pl.ds(gl, 16)])
        plsc.store_scatter(vb, [r, jnp.full((16,), 1, jnp.int32)],
                           py[gr, pl.ds(gl, 16)])
        plsc.store_scatter(vb, [r, jnp.full((16,), 2, jnp.int32)],
                           pz[gr, pl.ds(gl, 16)])
        plsc.store_scatter(vb, [r, jnp.full((16,), 3, jnp.int32)], ones16)
        return carry2

      lax.fori_loop(0, grp, rep, 0)
      for b in range(fire):
        pltpu.sync_copy(vb.at[pl.ds(b * CH, CH), :],
                        sp.at[iv.at[b]], add=True)
      return carry

    lax.fori_loop(0, no, body, 0)
    plsc.subcore_barrier()
    pltpu.sync_copy(sp.at[pl.ds(s * nrows, nrows), :],
                    out.at[c, pl.ds(s * nrows, nrows), :])

  return sk


def _make_scatter_feathalves(n, e, d2, interpret=False):
  fire = 4
  et = e // NS
  j2 = et // CH
  no = j2 // fire
  nrows = n // NS

  @functools.partial(
      pl.kernel,
      out_type=jax.ShapeDtypeStruct((NC, n, d2), jnp.float32),
      mesh=_mesh(),
      scratch_types=[
          pltpu.VMEM((fire, CH), jnp.int32),
          pltpu.VMEM((fire * CH, d2), jnp.float32),
          pltpu.VMEM_SHARED((n, d2), jnp.float32),
          pltpu.SemaphoreType.DMA,
          pltpu.SemaphoreType.DMA,
      ],
      compiler_params=_SC_PARAMS,
      interpret=interpret,
  )
  def sk(vals, idx, out, iv, vb, sp, sem, sem2):
    c = lax.axis_index("c")
    s = lax.axis_index("s")
    _zero_spmem(vb, sp, s, nrows, fire * CH, d2)
    plsc.subcore_barrier()
    base = s * et

    def body(o, carry):
      d1 = pltpu.async_copy(idx.at[s, pl.ds(o * fire, fire), :], iv, sem2)
      d2_ = pltpu.async_copy(
          vals.at[pl.ds(base + o * fire * CH, fire * CH), pl.ds(c * d2, d2)],
          vb, sem)
      d1.wait()
      d2_.wait()
      for b in range(fire):
        pltpu.sync_copy(vb.at[pl.ds(b * CH, CH), :],
                        sp.at[iv.at[b]], add=True)
      return carry

    lax.fori_loop(0, no, body, 0)
    plsc.subcore_barrier()
    pltpu.sync_copy(sp.at[pl.ds(s * nrows, nrows), :],
                    out.at[c, pl.ds(s * nrows, nrows), :])

  return sk


def _dot(a, b):
  return jnp.dot(a, b, preferred_element_type=jnp.float32)


def _embed_body(x_ref, fs_ref, fn_ref, w1_ref, b1_ref, w2_ref, b2_ref,
                wa_ref, wc_ref, h_ref, hra_ref, hcc_ref):
  xin = jnp.concatenate([x_ref[...], fs_ref[...], fn_ref[...]], axis=1)
  a = _swish(_dot(xin, w1_ref[...]) + b1_ref[...])
  h = _dot(a, w2_ref[...]) + b2_ref[...]
  h_ref[...] = h
  hra_ref[...] = _dot(h, wa_ref[...])
  hcc_ref[...] = _dot(h, wc_ref[...])


def _cos_body(evt_ref, lenq_ref, g_ref, cos_ref):
  g = g_ref[...]
  vsx = g[0] + g[4]
  vsy = g[1] + g[5]
  vsz = g[2] + g[6]
  cnt = g[3] + g[7]
  evx = evt_ref[0]
  evy = evt_ref[1]
  evz = evt_ref[2]
  dot = evx * vsx + evy * vsy + evz * vsz
  n2 = vsx * vsx + vsy * vsy + vsz * vsz
  den = lenq_ref[...] * (jnp.sqrt(n2) + 1e-6 * cnt)
  cos_ref[...] = jnp.clip(dot / den, -1.0, 1.0)


def _edge_body(inv_w2, hid, gh_ref, lenq_ref, cosq_ref, cen_ref, wr_ref,
               b1_ref, w2_ref, b2_ref, wt1_ref, wtc_ref, bt1_ref, wt2_ref,
               bt2_ref, m2_ref):
  d = lenq_ref[...]
  cos = cosq_ref[...]
  gh = gh_ref[...]
  rbf = jnp.exp(-((d - cen_ref[...]) ** 2) * inv_w2)
  z = gh[:, :hid] + gh[:, hid:] + _dot(rbf, wr_ref[...]) + b1_ref[...]
  m = _swish(_dot(_swish(z), w2_ref[...]) + b2_ref[...])
  tt = _swish(_dot(m, wt1_ref[...]) + cos * wtc_ref[...] + bt1_ref[...])
  g = jax.nn.sigmoid(_dot(tt, wt2_ref[...]) + bt2_ref[...])
  m2 = m * g
  m2_ref[...] = jnp.concatenate([m2, m2], axis=1)


def _node_body(h_ref, ag_ref, ag2_ref, wn1a_ref, wn1b_ref, bn1_ref, wn2_ref,
               bn2_ref, wa_ref, wc_ref, hn_ref, hra_ref, hcc_ref):
  h = h_ref[...]
  aggr = jnp.concatenate([ag_ref[0] + ag2_ref[0], ag_ref[1] + ag2_ref[1]],
                         axis=1)
  u = _swish(_dot(h, wn1a_ref[...]) + _dot(aggr, wn1b_ref[...]) + bn1_ref[...])
  hn = h + _swish(_dot(u, wn2_ref[...]) + bn2_ref[...])
  hn_ref[...] = hn
  hra_ref[...] = _dot(hn, wa_ref[...])
  hcc_ref[...] = _dot(hn, wc_ref[...])


def _node_out_body(h_ref, ag_ref, ag2_ref, wn1a_ref, wn1b_ref, bn1_ref,
                   wn2_ref, bn2_ref, wo1_ref, bo1_ref, wo2_ref, bo2_ref,
                   out_ref):
  h = h_ref[...]
  aggr = jnp.concatenate([ag_ref[0] + ag2_ref[0], ag_ref[1] + ag2_ref[1]],
                         axis=1)
  u = _swish(_dot(h, wn1a_ref[...]) + _dot(aggr, wn1b_ref[...]) + bn1_ref[...])
  hn = h + _swish(_dot(u, wn2_ref[...]) + bn2_ref[...])
  v = _swish(_dot(hn, wo1_ref[...]) + bo1_ref[...])
  out_ref[...] = jnp.tanh(_dot(v, wo2_ref[...]) + bo2_ref[...])


def _full(shape):
  return pl.BlockSpec(shape, lambda i: (0,) * len(shape))


def _rows(t, d):
  return pl.BlockSpec((t, d), lambda i: (i, 0))


def _rows3(lead, t, d):
  return pl.BlockSpec((lead, t, d), lambda i: (0, i, 0))


def kernel(x, forces_stack, forces_norm, edge_index, edge_length, edge_vec,
           params, interpret=False):
  f32 = jnp.float32
  n, dx = x.shape
  e = edge_index.shape[1]
  hid = params["emb"][1][0].shape[1]
  nrbf = params["blocks"][0]["edge"][0][0].shape[0] - 2 * hid
  inv_w2 = (nrbf / 5.0) ** 2
  centers = jnp.zeros((1, hid), f32).at[0, :nrbf].set(
      jnp.linspace(0.0, 5.0, nrbf, dtype=f32))

  n_pad = -(-n // (8 * NS)) * (8 * NS)
  ep = -(-e // (NW * CH * 8)) * (NW * CH * 8)
  pad = ep - e
  row = edge_index[0]
  col = edge_index[1]
  rowp = jnp.pad(row, (0, pad))
  colp = jnp.pad(col, (0, pad))
  cols = jnp.pad(col, (0, pad), constant_values=n_pad - 1)
  ew = ep // NW
  j = ew // CH
  row_g = rowp.reshape(NW, j, CH)
  col_g = colp.reshape(NW, j, CH)
  col_eh = cols.reshape(NC, NS, j, CH)
  col_fh = cols.reshape(NS, (ep // NS) // CH, CH)
  epq = ep // 128
  lenp = jnp.pad(edge_length, (0, pad))
  lenq = lenp.reshape(epq, 128)
  evx = jnp.pad(edge_vec[:, 0], (0, pad))
  evy = jnp.pad(edge_vec[:, 1], (0, pad))
  evz = jnp.pad(edge_vec[:, 2], (0, pad))
  evt3q = jnp.stack([evx, evy, evz]).reshape(3, epq, 128)

  te = _pick_tile(ep, 4096)
  tn = _pick_tile(n, 5000)
  tq = _pick_tile(epq, 640)
  ge = ep // te
  gn = n // tn

  def rb(w):
    return w.reshape(1, -1)

  blocks = []
  for bp in params["blocks"]:
    (we1, be1), (we2, be2) = bp["edge"]
    (wn1, bn1), (wn2, bn2) = bp["node"]
    (wt1, bt1), (wt2, bt2) = bp["three"]
    blocks.append(dict(
        wa=we1[:hid], wrbf=jnp.zeros((hid, hid), f32).at[:nrbf].set(
            we1[hid:hid + nrbf]), wc=we1[hid + nrbf:],
        be1=rb(be1), we2=we2, be2=rb(be2),
        wt1=wt1[:hid], wtc=wt1[hid:hid + 1], bt1=rb(bt1), wt2=wt2,
        bt2=rb(bt2),
        wn1a=wn1[:hid], wn1b=wn1[hid:], bn1=rb(bn1), wn2=wn2, bn2=rb(bn2)))

  (w1, b1), (w2, b2) = params["emb"]
  h, hra, hcc = pl.pallas_call(
      _embed_body,
      grid=(gn,),
      in_specs=[_rows(tn, dx), _rows(tn, 3), _rows(tn, 1),
                _full((dx + 4, hid)), _full((1, hid)), _full((hid, hid)),
                _full((1, hid)), _full((hid, hid)), _full((hid, hid))],
      out_specs=[_rows(tn, hid), _rows(tn, hid), _rows(tn, hid)],
      out_shape=[jax.ShapeDtypeStruct((n, hid), f32)] * 3,
      interpret=interpret,
  )(x, forces_stack, forces_norm, w1, rb(b1), w2, rb(b2),
    blocks[0]["wa"], blocks[0]["wc"])

  part = _make_scatter_ev(n_pad, ep, interpret)(
      evx.reshape(-1, 128), evy.reshape(-1, 128), evz.reshape(-1, 128), col_eh)
  g8q = _make_gather_planes(n_pad, ep, interpret)(
      part[0], part[1], col_g).reshape(8, epq, 128)
  cosq = pl.pallas_call(
      _cos_body,
      grid=(epq // tq,),
      in_specs=[_rows3(3, tq, 128), _rows(tq, 128), _rows3(8, tq, 128)],
      out_specs=_rows(tq, 128),
      out_shape=jax.ShapeDtypeStruct((epq, 128), f32),
      interpret=interpret,
  )(evt3q, lenq, g8q)

  len2d = lenp.reshape(ep, 1)
  cos2d = jnp.reshape(cosq, (ep, 1))

  eph = ep // 2
  geh = ge // 2
  jh = (eph // NW) // CH
  j2h = (eph // NS) // CH
  row_h = [rowp[:eph].reshape(NW, jh, CH), rowp[eph:].reshape(NW, jh, CH)]
  col_h = [colp[:eph].reshape(NW, jh, CH), colp[eph:].reshape(NW, jh, CH)]
  cs_h = [cols[:eph].reshape(NS, j2h, CH), cols[eph:].reshape(NS, j2h, CH)]
  gather_hh = _make_gather_pair(n, eph, hid, interpret)
  scatter_m2 = _make_scatter_feathalves(n_pad, eph, hid // 2, interpret)

  def edge_half(bw, gh, h_idx):
    def off(i):
      return (i + h_idx * geh, 0)
    return pl.pallas_call(
        functools.partial(_edge_body, inv_w2, hid),
        grid=(geh,),
        in_specs=[_rows(te, 2 * hid), pl.BlockSpec((te, 1), off),
                  pl.BlockSpec((te, 1), off),
                  _full((1, hid)), _full((hid, hid)), _full((1, hid)),
                  _full((hid, hid)), _full((1, hid)), _full((hid, hid)),
                  _full((1, hid)), _full((1, hid)), _full((hid, hid)),
                  _full((1, hid))],
        out_specs=_rows(te, 2 * hid),
        out_shape=jax.ShapeDtypeStruct((eph, 2 * hid), f32),
        interpret=interpret,
    )(gh, len2d, cos2d, centers, bw["wrbf"], bw["be1"], bw["we2"],
      bw["be2"], bw["wt1"], bw["wtc"], bw["bt1"], bw["wt2"], bw["bt2"])

  for k, bw in enumerate(blocks):
    gh0 = gather_hh(hra, row_h[0], hcc, col_h[0])
    gh1 = gather_hh(hra, row_h[1], hcc, col_h[1])
    m2d0 = edge_half(bw, gh0, 0)
    m2d1 = edge_half(bw, gh1, 1)
    aggr0 = scatter_m2(m2d0, cs_h[0])
    aggr1 = scatter_m2(m2d1, cs_h[1])

    if k + 1 < len(blocks):
      nxt = blocks[k + 1]
      h, hra, hcc = pl.pallas_call(
          _node_body,
          grid=(gn,),
          in_specs=[_rows(tn, hid), _rows3(NC, tn, hid // 2),
                    _rows3(NC, tn, hid // 2),
                    _full((hid, hid)), _full((hid, hid)), _full((1, hid)),
                    _full((hid, hid)), _full((1, hid)), _full((hid, hid)),
                    _full((hid, hid))],
          out_specs=[_rows(tn, hid)] * 3,
          out_shape=[jax.ShapeDtypeStruct((n, hid), f32)] * 3,
          interpret=interpret,
      )(h, aggr0, aggr1, bw["wn1a"], bw["wn1b"], bw["bn1"], bw["wn2"],
        bw["bn2"], nxt["wa"], nxt["wc"])
    else:
      (wo1, bo1), (wo2, bo2) = params["out"]
      nl4 = wo1.shape[1]
      out = pl.pallas_call(
          _node_out_body,
          grid=(gn,),
          in_specs=[_rows(tn, hid), _rows3(NC, tn, hid // 2),
                    _rows3(NC, tn, hid // 2),
                    _full((hid, hid)), _full((hid, hid)), _full((1, hid)),
                    _full((hid, hid)), _full((1, hid)), _full((hid, nl4)),
                    _full((1, nl4)), _full((nl4, 3)), _full((1, 3))],
          out_specs=_rows(tn, 3),
          out_shape=jax.ShapeDtypeStruct((n, 3), f32),
          interpret=interpret,
      )(h, aggr0, aggr1, bw["wn1a"], bw["wn1b"], bw["bn1"], bw["wn2"],
        bw["bn2"], wo1, rb(bo1), wo2, rb(bo2))
  return out


# speedup vs baseline: 1.2721x; 1.2721x over previous
import functools

import jax
import jax.numpy as jnp
from jax import lax
from jax.experimental import pallas as pl
from jax.experimental.pallas import tpu as pltpu
from jax.experimental.pallas import tpu_sc as plsc

NC = 2
NS = 16
NW = NC * NS
CH = 128
FIRE = 8


def _swish(v):
  return v * jax.nn.sigmoid(v)


def _pick_tile(total, target):
  for t in range(min(target, total), 0, -8):
    if total % t == 0:
      return t
  return total


def _mesh():
  return plsc.VectorSubcoreMesh(
      core_axis_name="c", subcore_axis_name="s",
      num_cores=NC, num_subcores=NS)


_SC_PARAMS = pltpu.CompilerParams(use_tc_tiling_on_sc=False,
                                 needs_layout_passes=False)


def _make_gather_pair(n, e, d, interpret=False):
  fire = 4
  ew = e // NW
  j = ew // CH
  no = j // fire

  @functools.partial(
      pl.kernel,
      out_type=jax.ShapeDtypeStruct((e, 2 * d), jnp.float32),
      mesh=_mesh(),
      scratch_types=[
          pltpu.VMEM((j, CH), jnp.int32),
          pltpu.VMEM((j, CH), jnp.int32),
          pltpu.VMEM((fire * CH, d), jnp.float32),
          pltpu.VMEM((fire * CH, d), jnp.float32),
          pltpu.SemaphoreType.DMA,
      ],
      compiler_params=_SC_PARAMS,
      interpret=interpret,
  )
  def gk(ta, ia, tb, ib, out, iva, ivb, ba, bb, sem):
    wid = lax.axis_index("s") * NC + lax.axis_index("c")
    base = wid * ew
    pltpu.sync_copy(ia.at[wid], iva)
    pltpu.sync_copy(ib.at[wid], ivb)

    def body(o, carry):
      descs = []
      for b in range(fire):
        descs.append(pltpu.async_copy(
            ta.at[iva.at[o * fire + b]], ba.at[pl.ds(b * CH, CH), :], sem))
        descs.append(pltpu.async_copy(
            tb.at[ivb.at[o * fire + b]], bb.at[pl.ds(b * CH, CH), :], sem))
      for dsc in descs:
        dsc.wait()
      r = base + o * (fire * CH)
      pltpu.sync_copy(ba, out.at[pl.ds(r, fire * CH), pl.ds(0, d)])
      pltpu.sync_copy(bb, out.at[pl.ds(r, fire * CH), pl.ds(d, d)])
      return carry

    lax.fori_loop(0, no, body, 0)

  return gk


def _make_gather_planes(n, e, interpret=False):
  fire = 8
  ew = e // NW
  j = ew // CH
  no = j // fire

  @functools.partial(
      pl.kernel,
      out_type=jax.ShapeDtypeStruct((8, e), jnp.float32),
      mesh=_mesh(),
      scratch_types=[
          pltpu.VMEM((j, CH), jnp.int32),
          pltpu.VMEM((fire * CH, 4), jnp.float32),
          pltpu.VMEM((fire * CH, 4), jnp.float32),
          pltpu.VMEM((8, fire * CH), jnp.float32),
          pltpu.SemaphoreType.DMA,
      ],
      compiler_params=_SC_PARAMS,
      interpret=interpret,
  )
  def gk(t0, t1, idx, out, iv, b0, b1, pb, sem):
    wid = lax.axis_index("s") * NC + lax.axis_index("c")
    base = wid * ew
    pltpu.sync_copy(idx.at[wid], iv)
    iota = lax.iota(jnp.int32, 16)

    def body(o, carry):
      descs = []
      for b in range(fire):
        descs.append(pltpu.async_copy(
            t0.at[iv.at[o * fire + b]], b0.at[pl.ds(b * CH, CH), :], sem))
        descs.append(pltpu.async_copy(
            t1.at[iv.at[o * fire + b]], b1.at[pl.ds(b * CH, CH), :], sem))
      for dsc in descs:
        dsc.wait()

      def rep(g, carry2):
        r = g * 16 + iota
        for p in range(4):
          cols = jnp.full((16,), p, jnp.int32)
          pb[p, pl.ds(g * 16, 16)] = plsc.load_gather(b0, [r, cols])
          pb[p + 4, pl.ds(g * 16, 16)] = plsc.load_gather(b1, [r, cols])
        return carry2

      lax.fori_loop(0, fire * CH // 16, rep, 0)
      for p in range(8):
        pltpu.sync_copy(pb.at[p],
                        out.at[p, pl.ds(base + o * fire * CH, fire * CH)])
      return carry

    lax.fori_loop(0, no, body, 0)

  return gk


def _zero_spmem(vb, sp, s, nrows, chunk, d2):
  def zb(i, carry):
    for q in range(d2 // 16):
      vb[i, pl.ds(q * 16, 16)] = jnp.zeros((16,), jnp.float32)
    return carry

  lax.fori_loop(0, chunk, zb, 0)
  full = nrows // chunk
  for kk in range(full):
    pltpu.sync_copy(vb, sp.at[pl.ds(s * nrows + kk * chunk, chunk), :])
  rem = nrows - full * chunk
  if rem:
    pltpu.sync_copy(vb.at[pl.ds(0, rem), :],
                    sp.at[pl.ds(s * nrows + full * chunk, rem), :])


def _make_scatter_ev(n, e, interpret=False):
  fire = 4
  et = e // NW
  j = et // CH
  no = j // fire
  nrows = n // NS
  grp = fire * CH // 16

  @functools.partial(
      pl.kernel,
      out_type=jax.ShapeDtypeStruct((NC, n, 4), jnp.float32),
      mesh=_mesh(),
      scratch_types=[
          pltpu.VMEM((fire, CH), jnp.int32),
          pltpu.VMEM((fire * CH // 128, 128), jnp.float32),
          pltpu.VMEM((fire * CH // 128, 128), jnp.float32),
          pltpu.VMEM((fire * CH // 128, 128), jnp.float32),
          pltpu.VMEM((fire * CH, 4), jnp.float32),
          pltpu.VMEM_SHARED((n, 4), jnp.float32),
          pltpu.SemaphoreType.DMA,
          pltpu.SemaphoreType.DMA,
      ],
      compiler_params=_SC_PARAMS,
      interpret=interpret,
  )
  def sk(evx, evy, evz, idx, out, iv, px, py, pz, vb, sp, sem, sem2):
    c = lax.axis_index("c")
    s = lax.axis_index("s")
    iota = lax.iota(jnp.int32, 16)
    ones16 = jnp.ones((16,), jnp.float32)
    zeros16 = jnp.zeros((16,), jnp.float32)

    def zb(g, carry):
      r = g * 16 + iota
      for p in range(4):
        plsc.store_scatter(vb, [r, jnp.full((16,), p, jnp.int32)], zeros16)
      return carry

    lax.fori_loop(0, fire * CH // 16, zb, 0)
    full = nrows // (fire * CH)
    for kk in range(full):
      pltpu.sync_copy(vb, sp.at[pl.ds(s * nrows + kk * fire * CH, fire * CH), :])
    rem = nrows - full * fire * CH
    if rem:
      pltpu.sync_copy(vb.at[pl.ds(0, rem), :],
                      sp.at[pl.ds(s * nrows + full * fire * CH, rem), :])
    plsc.subcore_barrier()
    base = (c * NS + s) * et

    def body(o, carry):
      q0 = (base + o * fire * CH) // 128
      qr = fire * CH // 128
      d1 = pltpu.async_copy(idx.at[c, s, pl.ds(o * fire, fire), :], iv, sem2)
      dx = pltpu.async_copy(evx.at[pl.ds(q0, qr), :], px, sem)
      dy = pltpu.async_copy(evy.at[pl.ds(q0, qr), :], py, sem)
      dz = pltpu.async_copy(evz.at[pl.ds(q0, qr), :], pz, sem)
      d1.wait()
      dx.wait()
      dy.wait()
      dz.wait()

      def rep(g, carry2):
        r = g * 16 + iota
        gr = g // 8
        gl = (g % 8) * 16
        plsc.store_scatter(vb, [r, jnp.full((16,), 0, jnp.int32)],
                           px[gr, pl.ds(gl, 16)])
        plsc.store_scatter(vb, [r, jnp.full((16,), 1, jnp.int32)],
                           py[gr, pl.ds(gl, 16)])
        plsc.store_scatter(vb, [r, jnp.full((16,), 2, jnp.int32)],
                           pz[gr, pl.ds(gl, 16)])
        plsc.store_scatter(vb, [r, jnp.full((16,), 3, jnp.int32)], ones16)
        return carry2

      lax.fori_loop(0, grp, rep, 0)
      for b in range(fire):
        pltpu.sync_copy(vb.at[pl.ds(b * CH, CH), :],
                        sp.at[iv.at[b]], add=True)
      return carry

    lax.fori_loop(0, no, body, 0)
    plsc.subcore_barrier()
    pltpu.sync_copy(sp.at[pl.ds(s * nrows, nrows), :],
                    out.at[c, pl.ds(s * nrows, nrows), :])

  return sk


def _make_scatter_feathalves(n, e, d2, interpret=False):
  fire = 4
  et = e // NS
  j2 = et // CH
  no = j2 // fire
  nrows = n // NS

  @functools.partial(
      pl.kernel,
      out_type=jax.ShapeDtypeStruct((NC, n, d2), jnp.float32),
      mesh=_mesh(),
      scratch_types=[
          pltpu.VMEM((fire, CH), jnp.int32),
          pltpu.VMEM((fire * CH, d2), jnp.float32),
          pltpu.VMEM_SHARED((n, d2), jnp.float32),
          pltpu.SemaphoreType.DMA,
          pltpu.SemaphoreType.DMA,
      ],
      compiler_params=_SC_PARAMS,
      interpret=interpret,
  )
  def sk(vals, idx, out, iv, vb, sp, sem, sem2):
    c = lax.axis_index("c")
    s = lax.axis_index("s")
    _zero_spmem(vb, sp, s, nrows, fire * CH, d2)
    plsc.subcore_barrier()
    base = s * et

    def body(o, carry):
      d1 = pltpu.async_copy(idx.at[s, pl.ds(o * fire, fire), :], iv, sem2)
      d2_ = pltpu.async_copy(
          vals.at[pl.ds(base + o * fire * CH, fire * CH), pl.ds(c * d2, d2)],
          vb, sem)
      d1.wait()
      d2_.wait()
      for b in range(fire):
        pltpu.sync_copy(vb.at[pl.ds(b * CH, CH), :],
                        sp.at[iv.at[b]], add=True)
      return carry

    lax.fori_loop(0, no, body, 0)
    plsc.subcore_barrier()
    pltpu.sync_copy(sp.at[pl.ds(s * nrows, nrows), :],
                    out.at[c, pl.ds(s * nrows, nrows), :])

  return sk


def _dot(a, b):
  return jnp.dot(a, b, preferred_element_type=jnp.float32)


def _embed_body(x_ref, fs_ref, fn_ref, w1_ref, b1_ref, w2_ref, b2_ref,
                wa_ref, wc_ref, h_ref, hra_ref, hcc_ref):
  xin = jnp.concatenate([x_ref[...], fs_ref[...], fn_ref[...]], axis=1)
  a = _swish(_dot(xin, w1_ref[...]) + b1_ref[...])
  h = _dot(a, w2_ref[...]) + b2_ref[...]
  h_ref[...] = h
  hra_ref[...] = _dot(h, wa_ref[...])
  hcc_ref[...] = _dot(h, wc_ref[...])


def _cos_body(evt_ref, lenq_ref, g_ref, cos_ref):
  g = g_ref[...]
  vsx = g[0] + g[4]
  vsy = g[1] + g[5]
  vsz = g[2] + g[6]
  cnt = g[3] + g[7]
  evx = evt_ref[0]
  evy = evt_ref[1]
  evz = evt_ref[2]
  dot = evx * vsx + evy * vsy + evz * vsz
  n2 = vsx * vsx + vsy * vsy + vsz * vsz
  den = lenq_ref[...] * (jnp.sqrt(n2) + 1e-6 * cnt)
  cos_ref[...] = jnp.clip(dot / den, -1.0, 1.0)


def _edge_body(inv_w2, hid, gh_ref, lc_ref, cen_ref, wr_ref,
               b1_ref, w2_ref, b2_ref, wt1_ref, wtc_ref, bt1_ref, wt2_ref,
               bt2_ref, m2_ref):
  lc = lc_ref[...]
  d = lc[:, 0:1]
  cos = lc[:, 1:2]
  gh = gh_ref[...]
  rbf = jnp.exp(-((d - cen_ref[...]) ** 2) * inv_w2)
  z = gh[:, :hid] + gh[:, hid:] + _dot(rbf, wr_ref[...]) + b1_ref[...]
  m = _swish(_dot(_swish(z), w2_ref[...]) + b2_ref[...])
  tt = _swish(_dot(m, wt1_ref[...]) + cos * wtc_ref[...] + bt1_ref[...])
  g = jax.nn.sigmoid(_dot(tt, wt2_ref[...]) + bt2_ref[...])
  m2 = m * g
  m2_ref[...] = jnp.concatenate([m2, m2], axis=1)


def _node_body(h_ref, ag_ref, wn1a_ref, wn1b_ref, bn1_ref, wn2_ref, bn2_ref,
               wa_ref, wc_ref, hn_ref, hra_ref, hcc_ref):
  h = h_ref[...]
  aggr = jnp.concatenate([ag_ref[0], ag_ref[1]], axis=1)
  u = _swish(_dot(h, wn1a_ref[...]) + _dot(aggr, wn1b_ref[...]) + bn1_ref[...])
  hn = h + _swish(_dot(u, wn2_ref[...]) + bn2_ref[...])
  hn_ref[...] = hn
  hra_ref[...] = _dot(hn, wa_ref[...])
  hcc_ref[...] = _dot(hn, wc_ref[...])


def _node_out_body(h_ref, ag_ref, wn1a_ref, wn1b_ref, bn1_ref, wn2_ref,
                   bn2_ref, wo1_ref, bo1_ref, wo2_ref, bo2_ref, out_ref):
  h = h_ref[...]
  aggr = jnp.concatenate([ag_ref[0], ag_ref[1]], axis=1)
  u = _swish(_dot(h, wn1a_ref[...]) + _dot(aggr, wn1b_ref[...]) + bn1_ref[...])
  hn = h + _swish(_dot(u, wn2_ref[...]) + bn2_ref[...])
  v = _swish(_dot(hn, wo1_ref[...]) + bo1_ref[...])
  out_ref[...] = jnp.tanh(_dot(v, wo2_ref[...]) + bo2_ref[...])


def _full(shape):
  return pl.BlockSpec(shape, lambda i: (0,) * len(shape))


def _rows(t, d):
  return pl.BlockSpec((t, d), lambda i: (i, 0))


def _rows3(lead, t, d):
  return pl.BlockSpec((lead, t, d), lambda i: (0, i, 0))


def kernel(x, forces_stack, forces_norm, edge_index, edge_length, edge_vec,
           params, interpret=False):
  f32 = jnp.float32
  n, dx = x.shape
  e = edge_index.shape[1]
  hid = params["emb"][1][0].shape[1]
  nrbf = params["blocks"][0]["edge"][0][0].shape[0] - 2 * hid
  inv_w2 = (nrbf / 5.0) ** 2
  centers = jnp.zeros((1, hid), f32).at[0, :nrbf].set(
      jnp.linspace(0.0, 5.0, nrbf, dtype=f32))

  n_pad = -(-n // (8 * NS)) * (8 * NS)
  ep = -(-e // (NW * CH * 4)) * (NW * CH * 4)
  pad = ep - e
  row = edge_index[0]
  col = edge_index[1]
  rowp = jnp.pad(row, (0, pad))
  colp = jnp.pad(col, (0, pad))
  cols = jnp.pad(col, (0, pad), constant_values=n_pad - 1)
  ew = ep // NW
  j = ew // CH
  row_g = rowp.reshape(NW, j, CH)
  col_g = colp.reshape(NW, j, CH)
  col_eh = cols.reshape(NC, NS, j, CH)
  col_fh = cols.reshape(NS, (ep // NS) // CH, CH)
  epq = ep // 128
  lenp = jnp.pad(edge_length, (0, pad))
  lenq = lenp.reshape(epq, 128)
  evx = jnp.pad(edge_vec[:, 0], (0, pad))
  evy = jnp.pad(edge_vec[:, 1], (0, pad))
  evz = jnp.pad(edge_vec[:, 2], (0, pad))
  evt3q = jnp.stack([evx, evy, evz]).reshape(3, epq, 128)

  te = _pick_tile(ep, 4096)
  tn = _pick_tile(n, 5000)
  tq = _pick_tile(epq, 640)
  ge = ep // te
  gn = n // tn

  def rb(w):
    return w.reshape(1, -1)

  blocks = []
  for bp in params["blocks"]:
    (we1, be1), (we2, be2) = bp["edge"]
    (wn1, bn1), (wn2, bn2) = bp["node"]
    (wt1, bt1), (wt2, bt2) = bp["three"]
    blocks.append(dict(
        wa=we1[:hid], wrbf=jnp.zeros((hid, hid), f32).at[:nrbf].set(
            we1[hid:hid + nrbf]), wc=we1[hid + nrbf:],
        be1=rb(be1), we2=we2, be2=rb(be2),
        wt1=wt1[:hid], wtc=wt1[hid:hid + 1], bt1=rb(bt1), wt2=wt2,
        bt2=rb(bt2),
        wn1a=wn1[:hid], wn1b=wn1[hid:], bn1=rb(bn1), wn2=wn2, bn2=rb(bn2)))

  (w1, b1), (w2, b2) = params["emb"]
  h, hra, hcc = pl.pallas_call(
      _embed_body,
      grid=(gn,),
      in_specs=[_rows(tn, dx), _rows(tn, 3), _rows(tn, 1),
                _full((dx + 4, hid)), _full((1, hid)), _full((hid, hid)),
                _full((1, hid)), _full((hid, hid)), _full((hid, hid))],
      out_specs=[_rows(tn, hid), _rows(tn, hid), _rows(tn, hid)],
      out_shape=[jax.ShapeDtypeStruct((n, hid), f32)] * 3,
      interpret=interpret,
  )(x, forces_stack, forces_norm, w1, rb(b1), w2, rb(b2),
    blocks[0]["wa"], blocks[0]["wc"])

  part = _make_scatter_ev(n_pad, ep, interpret)(
      evx.reshape(-1, 128), evy.reshape(-1, 128), evz.reshape(-1, 128), col_eh)
  g8q = _make_gather_planes(n_pad, ep, interpret)(
      part[0], part[1], col_g).reshape(8, epq, 128)
  cosq = pl.pallas_call(
      _cos_body,
      grid=(epq // tq,),
      in_specs=[_rows3(3, tq, 128), _rows(tq, 128), _rows3(8, tq, 128)],
      out_specs=_rows(tq, 128),
      out_shape=jax.ShapeDtypeStruct((epq, 128), f32),
      interpret=interpret,
  )(evt3q, lenq, g8q)

  lc2 = jnp.stack([lenp, cosq.reshape(ep)], axis=1)

  gather_hh = _make_gather_pair(n, ep, hid, interpret)
  scatter_m2 = _make_scatter_feathalves(n_pad, ep, hid // 2, interpret)
  for k, bw in enumerate(blocks):
    gh = gather_hh(hra, row_g, hcc, col_g)
    m2d = pl.pallas_call(
        functools.partial(_edge_body, inv_w2, hid),
        grid=(ge,),
        in_specs=[_rows(te, 2 * hid), _rows(te, 2),
                  _full((1, hid)), _full((hid, hid)), _full((1, hid)),
                  _full((hid, hid)), _full((1, hid)), _full((hid, hid)),
                  _full((1, hid)), _full((1, hid)), _full((hid, hid)),
                  _full((1, hid))],
        out_specs=_rows(te, 2 * hid),
        out_shape=jax.ShapeDtypeStruct((ep, 2 * hid), f32),
        interpret=interpret,
    )(gh, lc2, centers, bw["wrbf"], bw["be1"], bw["we2"],
      bw["be2"], bw["wt1"], bw["wtc"], bw["bt1"], bw["wt2"], bw["bt2"])
    aggrh = scatter_m2(m2d, col_fh)

    if k + 1 < len(blocks):
      nxt = blocks[k + 1]
      h, hra, hcc = pl.pallas_call(
          _node_body,
          grid=(gn,),
          in_specs=[_rows(tn, hid), _rows3(NC, tn, hid // 2),
                    _full((hid, hid)), _full((hid, hid)), _full((1, hid)),
                    _full((hid, hid)), _full((1, hid)), _full((hid, hid)),
                    _full((hid, hid))],
          out_specs=[_rows(tn, hid)] * 3,
          out_shape=[jax.ShapeDtypeStruct((n, hid), f32)] * 3,
          interpret=interpret,
      )(h, aggrh, bw["wn1a"], bw["wn1b"], bw["bn1"], bw["wn2"], bw["bn2"],
        nxt["wa"], nxt["wc"])
    else:
      (wo1, bo1), (wo2, bo2) = params["out"]
      nl4 = wo1.shape[1]
      out = pl.pallas_call(
          _node_out_body,
          grid=(gn,),
          in_specs=[_rows(tn, hid), _rows3(NC, tn, hid // 2),
                    _full((hid, hid)), _full((hid, hid)), _full((1, hid)),
                    _full((hid, hid)), _full((1, hid)), _full((hid, nl4)),
                    _full((1, nl4)), _full((nl4, 3)), _full((1, 3))],
          out_specs=_rows(tn, 3),
          out_shape=jax.ShapeDtypeStruct((n, 3), f32),
          interpret=interpret,
      )(h, aggrh, bw["wn1a"], bw["wn1b"], bw["bn1"], bw["wn2"], bw["bn2"],
        wo1, rb(bo1), wo2, rb(bo2))
  return out
